# baseline (speedup 1.0000x reference)
import functools

import jax
import jax.numpy as jnp
from jax import lax
from jax.experimental import pallas as pl
from jax.experimental.pallas import tpu as pltpu
from jax.experimental.pallas import tpu_sc as plsc

_B = 16384
_D = 128
_S = 5120
_NC, _NS, _L = 2, 16, 16
_NW = _NC * _NS
_RPW = _S // _NW
_HALF = _RPW // 2
_GH = _HALF // _L

_TCB = 1024


def _half(z_ref, g_v, idx_v, out_v, rbase):
    lanes = lax.iota(jnp.int32, _L)

    def group(gi):
        lrow = gi * _L + lanes
        crow = rbase + lrow
        idxvec = idx_v[pl.ds(rbase + gi * _L, _L)]
        jv = crow & (_D - 1)
        acc = [jnp.zeros((_L,), jnp.float32) for _ in range(4)]
        for j in range(_D):
            zc = plsc.load_gather(z_ref, [lrow, jv])
            gc = plsc.load_gather(g_v, [idxvec, jv])
            acc[j % 4] = acc[j % 4] + zc * gc
            jv = (jv + 1) & (_D - 1)
        out_v[pl.ds(rbase + gi * _L, _L)] = (acc[0] + acc[1]) + (acc[2] + acc[3])

    plsc.parallel_loop(0, _GH, 1)(group)


def _sc_body(z_hbm, y0_hbm, g_hbm, out_hbm,
             z0_v, z1_v, g_v, idx_v, out_v, sem_g, sem_y, sem_z0, sem_z1):
    wid = lax.axis_index("s") * _NC + lax.axis_index("c")
    base = wid * _RPW
    cp_g = pltpu.async_copy(g_hbm, g_v, sem_g)
    cp_y = pltpu.async_copy(y0_hbm.at[pl.ds(base, _RPW)], idx_v, sem_y)
    cp_z0 = pltpu.async_copy(z_hbm.at[pl.ds(base, _HALF)], z0_v, sem_z0)
    cp_z1 = pltpu.async_copy(z_hbm.at[pl.ds(base + _HALF, _HALF)], z1_v, sem_z1)
    cp_g.wait()
    cp_y.wait()
    cp_z0.wait()
    _half(z0_v, g_v, idx_v, out_v, 0)
    cp_z1.wait()
    _half(z1_v, g_v, idx_v, out_v, _HALF)
    pltpu.sync_copy(out_v, out_hbm.at[pl.ds(base, _RPW)])


@functools.cache
def _sc_call():
    return functools.partial(
        pl.kernel,
        out_type=jax.ShapeDtypeStruct((_S,), jnp.float32),
        mesh=plsc.VectorSubcoreMesh(
            core_axis_name="c", subcore_axis_name="s",
            num_cores=_NC, num_subcores=_NS),
        compiler_params=pltpu.CompilerParams(needs_layout_passes=False),
        scratch_types=[
            pltpu.VMEM((_HALF, _D), jnp.float32),
            pltpu.VMEM((_HALF, _D), jnp.float32),
            pltpu.VMEM((4, _D), jnp.float32),
            pltpu.VMEM((_RPW,), jnp.int32),
            pltpu.VMEM((_RPW,), jnp.float32),
            pltpu.SemaphoreType.DMA,
            pltpu.SemaphoreType.DMA,
            pltpu.SemaphoreType.DMA,
            pltpu.SemaphoreType.DMA,
        ],
    )(_sc_body)


def _tc_body(z_ref, y0_ref, g_ref, out_ref):
    p = lax.dot_general(z_ref[...], g_ref[...], (((1,), (1,)), ((), ())),
                        preferred_element_type=jnp.float32,
                        precision=lax.Precision.HIGHEST)
    idx = y0_ref[...]
    k4 = lax.broadcasted_iota(jnp.int32, (_TCB, 4), 1)
    sel = jnp.where(idx[:, None] == k4, p, 0.0)
    out_ref[...] = jnp.sum(sel, axis=1)


@functools.cache
def _tc_call():
    return pl.pallas_call(
        _tc_body,
        grid=((_B - _S) // _TCB,),
        in_specs=[
            pl.BlockSpec((_TCB, _D), lambda i: (_S // _TCB + i, 0)),
            pl.BlockSpec((_TCB,), lambda i: (_S // _TCB + i,)),
            pl.BlockSpec((4, _D), lambda i: (0, 0)),
        ],
        out_specs=pl.BlockSpec((_TCB,), lambda i: (i,)),
        out_shape=jax.ShapeDtypeStruct((_B - _S,), jnp.float32),
    )


def kernel(z, y, gamma):
    y0 = y[:, 0].astype(jnp.int32)
    out_sc = _sc_call()(z, y0, gamma)
    out_tc = _tc_call()(z, y0, gamma)
    return jnp.concatenate([out_sc, out_tc])


# speedup vs baseline: 1.2272x; 1.2272x over previous
import functools

import jax
import jax.numpy as jnp
from jax import lax
from jax.experimental import pallas as pl
from jax.experimental.pallas import tpu as pltpu
from jax.experimental.pallas import tpu_sc as plsc

_B = 16384
_D = 128
_S = 5120
_NC, _NS, _L = 2, 16, 16
_NW = _NC * _NS
_RPW = _S // _NW
_HALF = _RPW // 2
_GH = _HALF // _L

_TCB = 1024


def _half(z_ref, g_v, idx_v, out_v, rbase):
    lanes = lax.iota(jnp.int32, _L)

    def group(gi):
        lrow = gi * _L + lanes
        crow = rbase + lrow
        idxvec = idx_v[pl.ds(rbase + gi * _L, _L)]
        jv = crow & (_D - 1)
        acc = [jnp.zeros((_L,), jnp.float32) for _ in range(4)]
        for j in range(_D):
            zc = plsc.load_gather(z_ref, [lrow, jv])
            gc = plsc.load_gather(g_v, [idxvec, jv])
            acc[j % 4] = acc[j % 4] + zc * gc
            jv = (jv + 1) & (_D - 1)
        out_v[pl.ds(rbase + gi * _L, _L)] = (acc[0] + acc[1]) + (acc[2] + acc[3])

    plsc.parallel_loop(0, _GH, 1)(group)


def _sc_body(z_hbm, y0_hbm, g_hbm, out_hbm,
             z0_v, z1_v, g_v, idx_v, out_v, sem_g, sem_y, sem_z0, sem_z1):
    wid = lax.axis_index("s") * _NC + lax.axis_index("c")
    base = wid * _RPW
    cp_g = pltpu.async_copy(g_hbm, g_v, sem_g)
    cp_y = pltpu.async_copy(y0_hbm.at[pl.ds(base, _RPW)], idx_v, sem_y)
    cp_z0 = pltpu.async_copy(z_hbm.at[pl.ds(base, _HALF)], z0_v, sem_z0)
    cp_z1 = pltpu.async_copy(z_hbm.at[pl.ds(base + _HALF, _HALF)], z1_v, sem_z1)
    cp_g.wait()
    cp_y.wait()
    cp_z0.wait()
    _half(z0_v, g_v, idx_v, out_v, 0)
    cp_z1.wait()
    _half(z1_v, g_v, idx_v, out_v, _HALF)
    pltpu.sync_copy(out_v, out_hbm.at[pl.ds(base, _RPW)])


@functools.cache
def _sc_call():
    return functools.partial(
        pl.kernel,
        out_type=jax.ShapeDtypeStruct((_S,), jnp.float32),
        mesh=plsc.VectorSubcoreMesh(
            core_axis_name="c", subcore_axis_name="s",
            num_cores=_NC, num_subcores=_NS),
        compiler_params=pltpu.CompilerParams(needs_layout_passes=False),
        scratch_types=[
            pltpu.VMEM((_HALF, _D), jnp.float32),
            pltpu.VMEM((_HALF, _D), jnp.float32),
            pltpu.VMEM((4, _D), jnp.float32),
            pltpu.VMEM((_RPW,), jnp.int32),
            pltpu.VMEM((_RPW,), jnp.float32),
            pltpu.SemaphoreType.DMA,
            pltpu.SemaphoreType.DMA,
            pltpu.SemaphoreType.DMA,
            pltpu.SemaphoreType.DMA,
        ],
    )(_sc_body)


def _tc_body(z_ref, y0_ref, g_ref, out_ref):
    idxb = y0_ref[...][:, None]
    g = g_ref[...]
    grow = jnp.where(idxb == 0, g[0:1, :],
                     jnp.where(idxb == 1, g[1:2, :],
                               jnp.where(idxb == 2, g[2:3, :], g[3:4, :])))
    out_ref[...] = jnp.sum(z_ref[...] * grow, axis=1)


@functools.cache
def _tc_call():
    return pl.pallas_call(
        _tc_body,
        grid=((_B - _S) // _TCB,),
        in_specs=[
            pl.BlockSpec((_TCB, _D), lambda i: (_S // _TCB + i, 0)),
            pl.BlockSpec((_TCB,), lambda i: (_S // _TCB + i,)),
            pl.BlockSpec((4, _D), lambda i: (0, 0)),
        ],
        out_specs=pl.BlockSpec((_TCB,), lambda i: (i,)),
        out_shape=jax.ShapeDtypeStruct((_B - _S,), jnp.float32),
    )


def kernel(z, y, gamma):
    y0 = y[:, 0].astype(jnp.int32)
    out_sc = _sc_call()(z, y0, gamma)
    out_tc = _tc_call()(z, y0, gamma)
    return jnp.concatenate([out_sc, out_tc])


# speedup vs baseline: 1.2917x; 1.0526x over previous
import functools

import jax
import jax.numpy as jnp
from jax import lax
from jax.experimental import pallas as pl
from jax.experimental.pallas import tpu as pltpu
from jax.experimental.pallas import tpu_sc as plsc

_B = 16384
_D = 128
_S = 6144
_NC, _NS, _L = 2, 16, 16
_NW = _NC * _NS
_RPW = _S // _NW
_HALF = _RPW // 2
_GH = _HALF // _L

_TCB = 2048


def _half(z_ref, g_v, idx_v, out_v, rbase):
    lanes = lax.iota(jnp.int32, _L)

    def group(gi):
        lrow = gi * _L + lanes
        crow = rbase + lrow
        idxvec = idx_v[pl.ds(rbase + gi * _L, _L)]
        jv = crow & (_D - 1)
        acc = [jnp.zeros((_L,), jnp.float32) for _ in range(4)]
        for j in range(_D):
            zc = plsc.load_gather(z_ref, [lrow, jv])
            gc = plsc.load_gather(g_v, [idxvec, jv])
            acc[j % 4] = acc[j % 4] + zc * gc
            jv = (jv + 1) & (_D - 1)
        out_v[pl.ds(rbase + gi * _L, _L)] = (acc[0] + acc[1]) + (acc[2] + acc[3])

    plsc.parallel_loop(0, _GH, 1)(group)


def _sc_body(z_hbm, y0_hbm, g_hbm, out_hbm,
             z0_v, z1_v, g_v, idx_v, out_v, sem_g, sem_y, sem_z0, sem_z1):
    wid = lax.axis_index("s") * _NC + lax.axis_index("c")
    base = wid * _RPW
    cp_g = pltpu.async_copy(g_hbm, g_v, sem_g)
    cp_y = pltpu.async_copy(y0_hbm.at[pl.ds(base, _RPW)], idx_v, sem_y)
    cp_z0 = pltpu.async_copy(z_hbm.at[pl.ds(base, _HALF)], z0_v, sem_z0)
    cp_z1 = pltpu.async_copy(z_hbm.at[pl.ds(base + _HALF, _HALF)], z1_v, sem_z1)
    cp_g.wait()
    cp_y.wait()
    cp_z0.wait()
    _half(z0_v, g_v, idx_v, out_v, 0)
    cp_z1.wait()
    _half(z1_v, g_v, idx_v, out_v, _HALF)
    pltpu.sync_copy(out_v, out_hbm.at[pl.ds(base, _RPW)])


@functools.cache
def _sc_call():
    return functools.partial(
        pl.kernel,
        out_type=jax.ShapeDtypeStruct((_S,), jnp.float32),
        mesh=plsc.VectorSubcoreMesh(
            core_axis_name="c", subcore_axis_name="s",
            num_cores=_NC, num_subcores=_NS),
        compiler_params=pltpu.CompilerParams(needs_layout_passes=False),
        scratch_types=[
            pltpu.VMEM((_HALF, _D), jnp.float32),
            pltpu.VMEM((_HALF, _D), jnp.float32),
            pltpu.VMEM((4, _D), jnp.float32),
            pltpu.VMEM((_RPW,), jnp.int32),
            pltpu.VMEM((_RPW,), jnp.float32),
            pltpu.SemaphoreType.DMA,
            pltpu.SemaphoreType.DMA,
            pltpu.SemaphoreType.DMA,
            pltpu.SemaphoreType.DMA,
        ],
    )(_sc_body)


def _tc_body(z_ref, y0_ref, g_ref, out_ref):
    idxb = y0_ref[...][:, None]
    g = g_ref[...]
    grow = jnp.where(idxb == 0, g[0:1, :],
                     jnp.where(idxb == 1, g[1:2, :],
                               jnp.where(idxb == 2, g[2:3, :], g[3:4, :])))
    out_ref[...] = jnp.sum(z_ref[...] * grow, axis=1)


@functools.cache
def _tc_call():
    return pl.pallas_call(
        _tc_body,
        grid=((_B - _S) // _TCB,),
        in_specs=[
            pl.BlockSpec((_TCB, _D), lambda i: (_S // _TCB + i, 0)),
            pl.BlockSpec((_TCB,), lambda i: (_S // _TCB + i,)),
            pl.BlockSpec((4, _D), lambda i: (0, 0)),
        ],
        out_specs=pl.BlockSpec((_TCB,), lambda i: (i,)),
        out_shape=jax.ShapeDtypeStruct((_B - _S,), jnp.float32),
    )


def kernel(z, y, gamma):
    y0 = y[:, 0].astype(jnp.int32)
    out_sc = _sc_call()(z, y0, gamma)
    out_tc = _tc_call()(z, y0, gamma)
    return jnp.concatenate([out_sc, out_tc])
